# baseline (speedup 1.0000x reference)
import functools

import jax
import jax.numpy as jnp
from jax import lax
from jax.experimental import pallas as pl
from jax.experimental.pallas import tpu as pltpu
from jax.experimental.pallas import tpu_sc as plsc

_H = 256
_NAP = 10240
_NG = 64
_C = 128
_NW = 32
_F32 = jnp.float32


def _proj(x, Wcat, bcat, widths, bn):
    N, K = x.shape
    M = Wcat.shape[1]
    assert N % bn == 0
    offs = []
    o = 0
    for w in widths:
        offs.append(o)
        o += w
    assert o == M

    def body(x_ref, w_ref, b_ref, *out_refs):
        acc = jnp.dot(x_ref[...], w_ref[...], preferred_element_type=_F32)
        acc = acc + b_ref[...]
        for r, off, w in zip(out_refs, offs, widths):
            r[...] = acc[:, off:off + w]

    return pl.pallas_call(
        body,
        grid=(N // bn,),
        in_specs=[
            pl.BlockSpec((bn, K), lambda i: (i, 0)),
            pl.BlockSpec((K, M), lambda i: (0, 0)),
            pl.BlockSpec((1, M), lambda i: (0, 0)),
        ],
        out_specs=[pl.BlockSpec((bn, w), lambda i: (i, 0)) for w in widths],
        out_shape=[jax.ShapeDtypeStruct((N, w), _F32) for w in widths],
    )(x, Wcat, bcat)


def _den_reduce(denp):
    def body(p_ref, o_ref):
        o_ref[...] = jnp.sum(p_ref[...], axis=0)

    out = pl.pallas_call(
        body,
        out_shape=jax.ShapeDtypeStruct((8, _NAP // 8), _F32),
    )(denp.reshape(_NW, 8, _NAP // 8))
    return out.reshape(_NAP)


def _stats(aggcat1, aggcat2, ha, Ws1, bs1, Ws2, bs2):
    bn = 512
    ngrid = _NAP // bn

    def body(a10, a11, a20, a21, h, w1, b1, w2, b2, o1, o2, s_ref):
        i = pl.program_id(0)
        hblk = h[...]
        o1v = jnp.concatenate([a10[...], a11[...]], axis=1) \
            + jnp.dot(hblk, w1[...], preferred_element_type=_F32) + b1[...]
        o2v = jnp.concatenate([a20[...], a21[...]], axis=1) \
            + jnp.dot(hblk, w2[...], preferred_element_type=_F32) + b2[...]
        o1[...] = o1v
        o2[...] = o2v
        row = i * bn + lax.broadcasted_iota(jnp.int32, (bn, 1), 0)
        m1 = jnp.where(row < 10000, o1v, 0.0)
        m2 = jnp.where(row < 10000, o2v, 0.0)
        part = jnp.concatenate([
            jnp.sum(m1, axis=0, keepdims=True),
            jnp.sum(m1 * m1, axis=0, keepdims=True),
            jnp.sum(m2, axis=0, keepdims=True),
            jnp.sum(m2 * m2, axis=0, keepdims=True),
            jnp.zeros((4, _H), _F32),
        ], axis=0)

        @pl.when(i == 0)
        def _():
            s_ref[...] = jnp.zeros_like(s_ref)

        s_ref[...] += part

    return pl.pallas_call(
        body,
        grid=(ngrid,),
        in_specs=[
            pl.BlockSpec((bn, 128), lambda i: (i, 0)),
            pl.BlockSpec((bn, 128), lambda i: (i + ngrid, 0)),
            pl.BlockSpec((bn, 128), lambda i: (i, 0)),
            pl.BlockSpec((bn, 128), lambda i: (i + ngrid, 0)),
            pl.BlockSpec((bn, _H), lambda i: (i, 0)),
            pl.BlockSpec((_H, _H), lambda i: (0, 0)),
            pl.BlockSpec((1, _H), lambda i: (0, 0)),
            pl.BlockSpec((_H, _H), lambda i: (0, 0)),
            pl.BlockSpec((1, _H), lambda i: (0, 0)),
        ],
        out_specs=[
            pl.BlockSpec((bn, _H), lambda i: (i, 0)),
            pl.BlockSpec((bn, _H), lambda i: (i, 0)),
            pl.BlockSpec((8, _H), lambda i: (0, 0)),
        ],
        out_shape=[
            jax.ShapeDtypeStruct((_NAP, _H), _F32),
            jax.ShapeDtypeStruct((_NAP, _H), _F32),
            jax.ShapeDtypeStruct((8, _H), _F32),
        ],
    )(aggcat1, aggcat1, aggcat2, aggcat2, ha, Ws1, bs1, Ws2, bs2)


def _norm_combine(o1, o2, sums, g1, b1, g2, b2):
    bn = 512
    inv_n = 1.0 / 10000.0

    def body(o1_ref, o2_ref, s_ref, g1_ref, b1_ref, g2_ref, b2_ref, h_ref):
        i = pl.program_id(0)
        s = s_ref[...]
        mu1 = s[0:1, :] * inv_n
        var1 = s[1:2, :] * inv_n - mu1 * mu1
        mu2 = s[2:3, :] * inv_n
        var2 = s[3:4, :] * inv_n - mu2 * mu2
        n1 = (o1_ref[...] - mu1) * lax.rsqrt(var1 + 1e-5) * g1_ref[...] + b1_ref[...]
        n2 = (o2_ref[...] - mu2) * lax.rsqrt(var2 + 1e-5) * g2_ref[...] + b2_ref[...]
        h = jnp.maximum(0.5 * (n1 + n2), 0.0)
        row = i * bn + lax.broadcasted_iota(jnp.int32, (bn, 1), 0)
        h_ref[...] = jnp.where(row < 10000, h, 0.0)

    def out_map(i):
        return (i, 0)

    return pl.pallas_call(
        body,
        grid=(_NAP // bn,),
        in_specs=[
            pl.BlockSpec((bn, _H), lambda i: (i, 0)),
            pl.BlockSpec((bn, _H), lambda i: (i, 0)),
            pl.BlockSpec((8, _H), lambda i: (0, 0)),
            pl.BlockSpec((1, _H), lambda i: (0, 0)),
            pl.BlockSpec((1, _H), lambda i: (0, 0)),
            pl.BlockSpec((1, _H), lambda i: (0, 0)),
            pl.BlockSpec((1, _H), lambda i: (0, 0)),
        ],
        out_specs=pl.BlockSpec((bn, _H), out_map),
        out_shape=jax.ShapeDtypeStruct((_NAP, _H), _F32),
    )(o1, o2, sums, g1, b1, g2, b2)


def _pool(ha, batch3):
    bn = 400
    ngrid = 10000 // bn

    def body(h_ref, b_ref, p_ref, c_ref):
        i = pl.program_id(0)
        seg = b_ref[0, 0, :]
        onehot = (seg[None, :] == lax.broadcasted_iota(jnp.int32, (_NG, bn), 0)
                  ).astype(_F32)

        @pl.when(i == 0)
        def _():
            p_ref[...] = jnp.zeros_like(p_ref)
            c_ref[...] = jnp.zeros_like(c_ref)

        p_ref[...] += jnp.dot(onehot, h_ref[...], preferred_element_type=_F32)
        c_ref[...] += jnp.dot(onehot, jnp.ones((bn, _H), _F32),
                              preferred_element_type=_F32)

    return pl.pallas_call(
        body,
        grid=(ngrid,),
        in_specs=[
            pl.BlockSpec((bn, _H), lambda i: (i, 0)),
            pl.BlockSpec((1, 1, bn), lambda i: (i, 0, 0)),
        ],
        out_specs=[
            pl.BlockSpec((_NG, _H), lambda i: (0, 0)),
            pl.BlockSpec((_NG, _H), lambda i: (0, 0)),
        ],
        out_shape=[
            jax.ShapeDtypeStruct((_NG, _H), _F32),
            jax.ShapeDtypeStruct((_NG, _H), _F32),
        ],
    )(ha, batch3)


def _head(pooled, cnt2d, projW, projb, outWp, outbp):
    def body(p_ref, c_ref, w_ref, b_ref, w2_ref, b2_ref, o_ref):
        mean = p_ref[...] / jnp.maximum(c_ref[...], 1.0)
        x = jnp.dot(mean, w_ref[...], preferred_element_type=_F32) + b_ref[...]
        x = jnp.logaddexp(x, 0.0)
        o_ref[...] = jnp.dot(x, w2_ref[...], preferred_element_type=_F32) + b2_ref[...]

    return pl.pallas_call(
        body,
        out_shape=jax.ShapeDtypeStruct((_NG, 128), _F32),
    )(pooled, cnt2d, projW, projb, outWp, outbp)


def _sc_alpha_den(q, k, src, dst, Epad, sup0=None):
    CA = 64
    SB = 1024
    EperW = Epad // _NW
    nsup = EperW // SB
    npair = SB // (2 * CA)
    if sup0 is None:
        sup0 = nsup
    sup1 = 2 * nsup - sup0
    mesh = plsc.VectorSubcoreMesh(core_axis_name="c", subcore_axis_name="s")

    @functools.partial(
        pl.kernel,
        out_type=(
            jax.ShapeDtypeStruct((Epad + 128, ), _F32),
            jax.ShapeDtypeStruct((_NW, _NAP), _F32),
        ),
        mesh=mesh,
        compiler_params=pltpu.CompilerParams(
            use_tc_tiling_on_sc=False, needs_layout_passes=False),
        scratch_types=[
            pltpu.VMEM((SB + CA,), jnp.int32),
            pltpu.VMEM((SB + CA,), jnp.int32),
            pltpu.VMEM((SB,), _F32),
            pltpu.VMEM((CA, _H), _F32),
            pltpu.VMEM((CA, _H), _F32),
            pltpu.VMEM((CA, _H), _F32),
            pltpu.VMEM((CA, _H), _F32),
            pltpu.VMEM((16, 16), _F32),
            pltpu.VMEM((_NAP,), _F32),
            pltpu.SemaphoreType.DMA,
            pltpu.SemaphoreType.DMA,
        ],
    )
    def kern(q_h, k_h, src_h, dst_h, alpha_h, denp_h,
             src_v, dst_v, al_v, qr0, kr0, qr1, kr1, tbuf, den_v, sem0, sem1):
        cid = lax.axis_index("c")
        sid = lax.axis_index("s")
        wid = cid * 16 + sid
        lane = lax.iota(jnp.int32, 16)

        def zero_body(i, carry):
            den_v[pl.ds(i * 16, 16)] = jnp.zeros((16,), _F32)
            return carry
        lax.fori_loop(0, _NAP // 16, zero_body, 0)

        qbuf = (qr0, qr1)
        kbuf = (kr0, kr1)
        sems = (sem0, sem1)

        def fetch(b, off):
            pltpu.async_copy(q_h.at[dst_v.at[pl.ds(off, CA)]], qbuf[b], sems[b])
            pltpu.async_copy(k_h.at[src_v.at[pl.ds(off, CA)]], kbuf[b], sems[b])

        def wait(b):
            pltpu.make_async_copy(q_h.at[dst_v.at[pl.ds(0, CA)]], qbuf[b],
                                  sems[b]).wait()
            pltpu.make_async_copy(k_h.at[src_v.at[pl.ds(0, CA)]], kbuf[b],
                                  sems[b]).wait()

        def compute(b, off):
            qr = qbuf[b]
            kr = kbuf[b]

            def grp_body(g, c2):
                for l in range(16):
                    e = g * 16 + l
                    p = qr[e, pl.ds(0, 16)] * kr[e, pl.ds(0, 16)]
                    for j in range(1, 16):
                        p = p + qr[e, pl.ds(j * 16, 16)] * kr[e, pl.ds(j * 16, 16)]
                    plsc.store_scatter(tbuf, [lane, jnp.full((16,), l, jnp.int32)], p)
                s = tbuf[0, pl.ds(0, 16)]
                for r in range(1, 16):
                    s = s + tbuf[r, pl.ds(0, 16)]
                a16 = s * 0.0625
                al_v[pl.ds(off + g * 16, 16)] = a16
                e16 = jnp.exp(a16)
                d16 = dst_v[pl.ds(off + g * 16, 16)]
                for l in range(16):
                    plsc.addupdate_scatter(den_v, [d16], e16, mask=lane == l)
                return c2
            lax.fori_loop(0, CA // 16, grp_body, 0)

        nsup_w = sup0 + cid * (sup1 - sup0)
        w0 = (cid * 16 * sup0 + sid * nsup_w) * SB

        def sup_body(si, carry):
            sbase = w0 + si * SB
            pltpu.sync_copy(src_h.at[pl.ds(sbase, SB + CA)], src_v)
            pltpu.sync_copy(dst_h.at[pl.ds(sbase, SB + CA)], dst_v)
            fetch(0, 0)

            def pair_body(pi, c2):
                off = (2 * pi) * CA
                fetch(1, off + CA)
                wait(0)
                compute(0, off)
                fetch(0, off + 2 * CA)
                wait(1)
                compute(1, off + CA)
                return c2
            lax.fori_loop(0, npair, pair_body, 0)
            wait(0)
            pltpu.sync_copy(al_v, alpha_h.at[pl.ds(sbase, SB)])
            return carry
        lax.fori_loop(0, nsup_w, sup_body, 0)

        pltpu.sync_copy(den_v, denp_h.at[wid])

    return kern(q, k, src, dst)


def _sc_agg(vcat, src, dst, alpha, den, zeros128, Epad, nsrc):
    CC = 64
    SB = 1024
    EperS = Epad // 16
    nsup = EperS // SB
    npair = SB // (2 * CC)
    rps = _NAP // 16
    mesh = plsc.VectorSubcoreMesh(core_axis_name="c", subcore_axis_name="s")

    @functools.partial(
        pl.kernel,
        out_type=jax.ShapeDtypeStruct((2 * _NAP, 128), _F32),
        mesh=mesh,
        compiler_params=pltpu.CompilerParams(
            use_tc_tiling_on_sc=False, needs_layout_passes=False),
        scratch_types=[
            pltpu.VMEM((SB + CC,), jnp.int32),
            pltpu.VMEM((SB + CC,), jnp.int32),
            pltpu.VMEM((SB,), _F32),
            pltpu.VMEM((2, CC), jnp.int32),
            pltpu.VMEM((CC, 128), _F32),
            pltpu.VMEM((CC, 128), _F32),
            pltpu.VMEM((CC, 128), _F32),
            pltpu.VMEM((CC, 128), _F32),
            pltpu.VMEM((_NAP,), _F32),
            pltpu.VMEM_SHARED((_NAP, 128), _F32),
            pltpu.SemaphoreType.DMA,
            pltpu.SemaphoreType.DMA,
            pltpu.SemaphoreType.DMA,
            pltpu.SemaphoreType.DMA,
        ],
    )
    def kern(vcat_h, src_h, dst_h, alpha_h, den_h, zero_h, out_h,
             src_v, dst_v, al_v, dst_sc, vr0, vr1, sc0, sc1, den_v, agg_sh,
             sem0, sem1, ssem0, ssem1):
        cid = lax.axis_index("c")
        sid = lax.axis_index("s")
        voff = cid * nsrc
        lane = lax.iota(jnp.int32, 16)

        for r in range(rps // 128):
            pltpu.sync_copy(zero_h, agg_sh.at[pl.ds(sid * rps + r * 128, 128)])
        pltpu.sync_copy(den_h, den_v)
        plsc.subcore_barrier()

        vbuf = (vr0, vr1)
        sems = (sem0, sem1)
        scbuf = (sc0, sc1)
        ssems = (ssem0, ssem1)

        def fetch(b, off):
            pltpu.async_copy(vcat_h.at[src_v.at[pl.ds(off, CC)]], vbuf[b],
                             sems[b])

        def wait(b):
            pltpu.make_async_copy(vcat_h.at[src_v.at[pl.ds(0, CC)]], vbuf[b],
                                  sems[b]).wait()

        def compute(b, off):
            pltpu.make_async_copy(scbuf[b], agg_sh.at[dst_sc.at[b]],
                                  ssems[b]).wait()
            vr = vbuf[b]
            sc_buf = scbuf[b]

            def grp_body(g, c2):
                a16 = al_v[pl.ds(off + g * 16, 16)]
                d16 = dst_v[pl.ds(off + g * 16, 16)]
                dst_sc[b, pl.ds(g * 16, 16)] = d16
                dg = plsc.load_gather(den_v, [d16])
                c16 = jnp.exp(a16) / (dg + 1e-16)
                for l in range(16):
                    e = g * 16 + l
                    cb = jnp.full((16,), c16[l], _F32)
                    for j in range(8):
                        sc_buf[e, pl.ds(j * 16, 16)] = vr[e, pl.ds(j * 16, 16)] * cb
                return c2
            lax.fori_loop(0, CC // 16, grp_body, 0)
            pltpu.async_copy(sc_buf, agg_sh.at[dst_sc.at[b]], ssems[b], add=True)

        def zb_body(i, carry):
            for j in range(8):
                sc0[i, pl.ds(j * 16, 16)] = jnp.zeros((16,), _F32)
                sc1[i, pl.ds(j * 16, 16)] = jnp.zeros((16,), _F32)
            return carry
        lax.fori_loop(0, CC, zb_body, 0)

        def zd_body(g, carry):
            dst_sc[0, pl.ds(g * 16, 16)] = jnp.zeros((16,), jnp.int32)
            dst_sc[1, pl.ds(g * 16, 16)] = jnp.zeros((16,), jnp.int32)
            return carry
        lax.fori_loop(0, CC // 16, zd_body, 0)
        pltpu.async_copy(sc0, agg_sh.at[dst_sc.at[0]], ssem0, add=True)
        pltpu.async_copy(sc1, agg_sh.at[dst_sc.at[1]], ssem1, add=True)

        s0 = sid * EperS

        def sup_body(si, carry):
            sbase = s0 + si * SB
            pltpu.sync_copy(src_h.at[pl.ds(sbase, SB + CC)], src_v)
            pltpu.sync_copy(dst_h.at[pl.ds(sbase, SB + CC)], dst_v)
            pltpu.sync_copy(alpha_h.at[pl.ds(sbase, SB)], al_v)

            def voff_body(g, c2):
                src_v[pl.ds(g * 16, 16)] = src_v[pl.ds(g * 16, 16)] + voff
                return c2
            lax.fori_loop(0, (SB + CC) // 16, voff_body, 0)

            fetch(0, 0)

            def pair_body(pi, c2):
                off = (2 * pi) * CC
                fetch(1, off + CC)
                wait(0)
                compute(0, off)
                fetch(0, off + 2 * CC)
                wait(1)
                compute(1, off + CC)
                return c2
            lax.fori_loop(0, npair, pair_body, 0)
            wait(0)
            return carry
        lax.fori_loop(0, nsup, sup_body, 0)

        pltpu.make_async_copy(sc0, agg_sh.at[dst_sc.at[0]], ssem0).wait()
        pltpu.make_async_copy(sc1, agg_sh.at[dst_sc.at[1]], ssem1).wait()

        plsc.subcore_barrier()
        pltpu.sync_copy(agg_sh.at[pl.ds(sid * rps, rps)],
                        out_h.at[pl.ds(cid * _NAP + sid * rps, rps)])

    return kern(vcat, src, dst, alpha, den, zeros128)


def _pad_edges(ei, Epad):
    E = ei.shape[1]
    src = jnp.concatenate([ei[0], jnp.zeros((Epad + 128 - E,), jnp.int32)])
    dst = jnp.concatenate([ei[1], jnp.full((Epad + 128 - E,), _NAP - 1, jnp.int32)])
    return src, dst


def _relation(h_src, h_dst_q, src, dst, Epad, Wq, bq, Wk, bk, Wv, bv, zeros128,
              bn_src, sup0=None):
    q, = _proj(h_dst_q, Wq, bq.reshape(1, _H), [_H], 1024)
    k, v0, v1 = _proj(h_src, jnp.concatenate([Wk, Wv], axis=1),
                      jnp.concatenate([bk, bv]).reshape(1, 2 * _H),
                      [_H, 128, 128], bn_src)
    vcat = jnp.concatenate([v0, v1], axis=0)
    alpha, denp = _sc_alpha_den(q, k, src, dst, Epad, sup0)
    den = _den_reduce(denp)
    return _sc_agg(vcat, src, dst, alpha, den, zeros128, Epad, h_src.shape[0])


def kernel(x_atom, x_bond, lin_aW, lin_ab, lin_bW, lin_bb, Wq, bq, Wk, bk,
           Wv, bv, Ws, bs, bn_g, bn_b, projW, projb, outW, outb,
           ei_ba, ei_aa, batch):
    NA = 10000
    xa = jnp.pad(x_atom, ((0, _NAP - NA), (0, 128 - x_atom.shape[1])))
    xb = jnp.pad(x_bond, ((0, 0), (0, 128 - x_bond.shape[1])))
    lin_aWp = jnp.pad(lin_aW, ((0, 128 - lin_aW.shape[0]), (0, 0)))
    lin_bWp = jnp.pad(lin_bW, ((0, 128 - lin_bW.shape[0]), (0, 0)))
    src_ba, dst_ba = _pad_edges(ei_ba, 32768)
    src_aa, dst_aa = _pad_edges(ei_aa, 163840)
    batch3 = batch.reshape(25, 1, 400)
    zeros128 = jnp.zeros((128, 128), _F32)
    outWp = jnp.pad(outW, ((0, 0), (0, 127)))
    outbp = jnp.pad(outb.reshape(1, 1), ((0, 0), (0, 127)))

    h_a, = _proj(xa, lin_aWp, lin_ab.reshape(1, _H), [_H], 1024)
    h_b, = _proj(xb, lin_bWp, lin_bb.reshape(1, _H), [_H], 1000)

    for l in range(3):
        aggcat1 = _relation(h_b, h_a, src_ba, dst_ba, 32768,
                            Wq[l, 0], bq[l, 0], Wk[l, 0], bk[l, 0],
                            Wv[l, 0], bv[l, 0], zeros128, 1000)
        aggcat2 = _relation(h_a, h_a, src_aa, dst_aa, 163840,
                            Wq[l, 1], bq[l, 1], Wk[l, 1], bk[l, 1],
                            Wv[l, 1], bv[l, 1], zeros128, 1024, sup0=9)
        o1, o2, sums = _stats(aggcat1, aggcat2, h_a,
                              Ws[l, 0], bs[l, 0].reshape(1, _H),
                              Ws[l, 1], bs[l, 1].reshape(1, _H))
        h_a = _norm_combine(o1, o2, sums,
                            bn_g[l, 0].reshape(1, _H), bn_b[l, 0].reshape(1, _H),
                            bn_g[l, 1].reshape(1, _H), bn_b[l, 1].reshape(1, _H))

    pooled, cnt2d = _pool(h_a, batch3)
    y = _head(pooled, cnt2d, projW, projb.reshape(1, _H), outWp, outbp)
    return y[:, :1]


# speedup vs baseline: 1.0213x; 1.0213x over previous
import functools

import jax
import jax.numpy as jnp
from jax import lax
from jax.experimental import pallas as pl
from jax.experimental.pallas import tpu as pltpu
from jax.experimental.pallas import tpu_sc as plsc

_H = 256
_NAP = 10240
_NG = 64
_C = 128
_NW = 32
_F32 = jnp.float32


def _proj(x, Wcat, bcat, widths, bn):
    N, K = x.shape
    M = Wcat.shape[1]
    assert N % bn == 0
    offs = []
    o = 0
    for w in widths:
        offs.append(o)
        o += w
    assert o == M

    def body(x_ref, w_ref, b_ref, *out_refs):
        acc = jnp.dot(x_ref[...], w_ref[...], preferred_element_type=_F32)
        acc = acc + b_ref[...]
        for r, off, w in zip(out_refs, offs, widths):
            r[...] = acc[:, off:off + w]

    return pl.pallas_call(
        body,
        grid=(N // bn,),
        in_specs=[
            pl.BlockSpec((bn, K), lambda i: (i, 0)),
            pl.BlockSpec((K, M), lambda i: (0, 0)),
            pl.BlockSpec((1, M), lambda i: (0, 0)),
        ],
        out_specs=[pl.BlockSpec((bn, w), lambda i: (i, 0)) for w in widths],
        out_shape=[jax.ShapeDtypeStruct((N, w), _F32) for w in widths],
    )(x, Wcat, bcat)


def _den_reduce(denp):
    def body(p_ref, o_ref):
        o_ref[...] = jnp.sum(p_ref[...], axis=0)

    out = pl.pallas_call(
        body,
        out_shape=jax.ShapeDtypeStruct((8, _NAP // 8), _F32),
    )(denp.reshape(_NW, 8, _NAP // 8))
    return out.reshape(_NAP)


def _stats(aggcat1, aggcat2, ha, Ws1, bs1, Ws2, bs2):
    bn = 512
    ngrid = _NAP // bn

    def body(a10, a11, a20, a21, h, w1, b1, w2, b2, o1, o2, s_ref):
        i = pl.program_id(0)
        hblk = h[...]
        o1v = jnp.concatenate([a10[...], a11[...]], axis=1) \
            + jnp.dot(hblk, w1[...], preferred_element_type=_F32) + b1[...]
        o2v = jnp.concatenate([a20[...], a21[...]], axis=1) \
            + jnp.dot(hblk, w2[...], preferred_element_type=_F32) + b2[...]
        o1[...] = o1v
        o2[...] = o2v
        row = i * bn + lax.broadcasted_iota(jnp.int32, (bn, 1), 0)
        m1 = jnp.where(row < 10000, o1v, 0.0)
        m2 = jnp.where(row < 10000, o2v, 0.0)
        part = jnp.concatenate([
            jnp.sum(m1, axis=0, keepdims=True),
            jnp.sum(m1 * m1, axis=0, keepdims=True),
            jnp.sum(m2, axis=0, keepdims=True),
            jnp.sum(m2 * m2, axis=0, keepdims=True),
            jnp.zeros((4, _H), _F32),
        ], axis=0)

        @pl.when(i == 0)
        def _():
            s_ref[...] = jnp.zeros_like(s_ref)

        s_ref[...] += part

    return pl.pallas_call(
        body,
        grid=(ngrid,),
        in_specs=[
            pl.BlockSpec((bn, 128), lambda i: (i, 0)),
            pl.BlockSpec((bn, 128), lambda i: (i + ngrid, 0)),
            pl.BlockSpec((bn, 128), lambda i: (i, 0)),
            pl.BlockSpec((bn, 128), lambda i: (i + ngrid, 0)),
            pl.BlockSpec((bn, _H), lambda i: (i, 0)),
            pl.BlockSpec((_H, _H), lambda i: (0, 0)),
            pl.BlockSpec((1, _H), lambda i: (0, 0)),
            pl.BlockSpec((_H, _H), lambda i: (0, 0)),
            pl.BlockSpec((1, _H), lambda i: (0, 0)),
        ],
        out_specs=[
            pl.BlockSpec((bn, _H), lambda i: (i, 0)),
            pl.BlockSpec((bn, _H), lambda i: (i, 0)),
            pl.BlockSpec((8, _H), lambda i: (0, 0)),
        ],
        out_shape=[
            jax.ShapeDtypeStruct((_NAP, _H), _F32),
            jax.ShapeDtypeStruct((_NAP, _H), _F32),
            jax.ShapeDtypeStruct((8, _H), _F32),
        ],
    )(aggcat1, aggcat1, aggcat2, aggcat2, ha, Ws1, bs1, Ws2, bs2)


def _norm_combine(o1, o2, sums, g1, b1, g2, b2):
    bn = 512
    inv_n = 1.0 / 10000.0

    def body(o1_ref, o2_ref, s_ref, g1_ref, b1_ref, g2_ref, b2_ref, h_ref):
        i = pl.program_id(0)
        s = s_ref[...]
        mu1 = s[0:1, :] * inv_n
        var1 = s[1:2, :] * inv_n - mu1 * mu1
        mu2 = s[2:3, :] * inv_n
        var2 = s[3:4, :] * inv_n - mu2 * mu2
        n1 = (o1_ref[...] - mu1) * lax.rsqrt(var1 + 1e-5) * g1_ref[...] + b1_ref[...]
        n2 = (o2_ref[...] - mu2) * lax.rsqrt(var2 + 1e-5) * g2_ref[...] + b2_ref[...]
        h = jnp.maximum(0.5 * (n1 + n2), 0.0)
        row = i * bn + lax.broadcasted_iota(jnp.int32, (bn, 1), 0)
        h_ref[...] = jnp.where(row < 10000, h, 0.0)

    def out_map(i):
        return (i, 0)

    return pl.pallas_call(
        body,
        grid=(_NAP // bn,),
        in_specs=[
            pl.BlockSpec((bn, _H), lambda i: (i, 0)),
            pl.BlockSpec((bn, _H), lambda i: (i, 0)),
            pl.BlockSpec((8, _H), lambda i: (0, 0)),
            pl.BlockSpec((1, _H), lambda i: (0, 0)),
            pl.BlockSpec((1, _H), lambda i: (0, 0)),
            pl.BlockSpec((1, _H), lambda i: (0, 0)),
            pl.BlockSpec((1, _H), lambda i: (0, 0)),
        ],
        out_specs=pl.BlockSpec((bn, _H), out_map),
        out_shape=jax.ShapeDtypeStruct((_NAP, _H), _F32),
    )(o1, o2, sums, g1, b1, g2, b2)


def _pool(ha, batch3):
    bn = 400
    ngrid = 10000 // bn

    def body(h_ref, b_ref, p_ref, c_ref):
        i = pl.program_id(0)
        seg = b_ref[0, 0, :]
        onehot = (seg[None, :] == lax.broadcasted_iota(jnp.int32, (_NG, bn), 0)
                  ).astype(_F32)

        @pl.when(i == 0)
        def _():
            p_ref[...] = jnp.zeros_like(p_ref)
            c_ref[...] = jnp.zeros_like(c_ref)

        p_ref[...] += jnp.dot(onehot, h_ref[...], preferred_element_type=_F32)
        c_ref[...] += jnp.dot(onehot, jnp.ones((bn, _H), _F32),
                              preferred_element_type=_F32)

    return pl.pallas_call(
        body,
        grid=(ngrid,),
        in_specs=[
            pl.BlockSpec((bn, _H), lambda i: (i, 0)),
            pl.BlockSpec((1, 1, bn), lambda i: (i, 0, 0)),
        ],
        out_specs=[
            pl.BlockSpec((_NG, _H), lambda i: (0, 0)),
            pl.BlockSpec((_NG, _H), lambda i: (0, 0)),
        ],
        out_shape=[
            jax.ShapeDtypeStruct((_NG, _H), _F32),
            jax.ShapeDtypeStruct((_NG, _H), _F32),
        ],
    )(ha, batch3)


def _head(pooled, cnt2d, projW, projb, outWp, outbp):
    def body(p_ref, c_ref, w_ref, b_ref, w2_ref, b2_ref, o_ref):
        mean = p_ref[...] / jnp.maximum(c_ref[...], 1.0)
        x = jnp.dot(mean, w_ref[...], preferred_element_type=_F32) + b_ref[...]
        x = jnp.logaddexp(x, 0.0)
        o_ref[...] = jnp.dot(x, w2_ref[...], preferred_element_type=_F32) + b2_ref[...]

    return pl.pallas_call(
        body,
        out_shape=jax.ShapeDtypeStruct((_NG, 128), _F32),
    )(pooled, cnt2d, projW, projb, outWp, outbp)


def _sc_alpha_den(q, k, src, dst, Epad, sup0=None):
    CA = 64
    SB = 1024
    EperW = Epad // _NW
    nsup = EperW // SB
    npair = SB // (2 * CA)
    if sup0 is None:
        sup0 = nsup
    sup1 = 2 * nsup - sup0
    mesh = plsc.VectorSubcoreMesh(core_axis_name="c", subcore_axis_name="s")

    @functools.partial(
        pl.kernel,
        out_type=(
            jax.ShapeDtypeStruct((Epad + 128, ), _F32),
            jax.ShapeDtypeStruct((_NW, _NAP), _F32),
        ),
        mesh=mesh,
        compiler_params=pltpu.CompilerParams(
            use_tc_tiling_on_sc=False, needs_layout_passes=False),
        scratch_types=[
            pltpu.VMEM((SB + CA,), jnp.int32),
            pltpu.VMEM((SB + CA,), jnp.int32),
            pltpu.VMEM((SB,), _F32),
            pltpu.VMEM((CA, _H), _F32),
            pltpu.VMEM((CA, _H), _F32),
            pltpu.VMEM((CA, _H), _F32),
            pltpu.VMEM((CA, _H), _F32),
            pltpu.VMEM((16, 16), _F32),
            pltpu.VMEM((_NAP,), _F32),
            pltpu.SemaphoreType.DMA,
            pltpu.SemaphoreType.DMA,
        ],
    )
    def kern(q_h, k_h, src_h, dst_h, alpha_h, denp_h,
             src_v, dst_v, al_v, qr0, kr0, qr1, kr1, tbuf, den_v, sem0, sem1):
        cid = lax.axis_index("c")
        sid = lax.axis_index("s")
        wid = cid * 16 + sid
        lane = lax.iota(jnp.int32, 16)

        def zero_body(i, carry):
            den_v[pl.ds(i * 16, 16)] = jnp.zeros((16,), _F32)
            return carry
        lax.fori_loop(0, _NAP // 16, zero_body, 0)

        qbuf = (qr0, qr1)
        kbuf = (kr0, kr1)
        sems = (sem0, sem1)

        def fetch(b, off):
            pltpu.async_copy(q_h.at[dst_v.at[pl.ds(off, CA)]], qbuf[b], sems[b])
            pltpu.async_copy(k_h.at[src_v.at[pl.ds(off, CA)]], kbuf[b], sems[b])

        def wait(b):
            pltpu.make_async_copy(q_h.at[dst_v.at[pl.ds(0, CA)]], qbuf[b],
                                  sems[b]).wait()
            pltpu.make_async_copy(k_h.at[src_v.at[pl.ds(0, CA)]], kbuf[b],
                                  sems[b]).wait()

        def compute(b, off):
            qr = qbuf[b]
            kr = kbuf[b]

            def grp_body(g, c2):
                for l in range(16):
                    e = g * 16 + l
                    p = qr[e, pl.ds(0, 16)] * kr[e, pl.ds(0, 16)]
                    for j in range(1, 16):
                        p = p + qr[e, pl.ds(j * 16, 16)] * kr[e, pl.ds(j * 16, 16)]
                    plsc.store_scatter(tbuf, [lane, jnp.full((16,), l, jnp.int32)], p)
                s = tbuf[0, pl.ds(0, 16)]
                for r in range(1, 16):
                    s = s + tbuf[r, pl.ds(0, 16)]
                a16 = s * 0.0625
                al_v[pl.ds(off + g * 16, 16)] = a16
                e16 = jnp.exp(a16)
                d16 = dst_v[pl.ds(off + g * 16, 16)]
                for l in range(16):
                    plsc.addupdate_scatter(den_v, [d16], e16, mask=lane == l)
                return c2
            lax.fori_loop(0, CA // 16, grp_body, 0)

        nsup_w = sup0 + cid * (sup1 - sup0)
        w0 = (cid * 16 * sup0 + sid * nsup_w) * SB

        def sup_body(si, carry):
            sbase = w0 + si * SB
            pltpu.sync_copy(src_h.at[pl.ds(sbase, SB + CA)], src_v)
            pltpu.sync_copy(dst_h.at[pl.ds(sbase, SB + CA)], dst_v)
            fetch(0, 0)

            def pair_body(pi, c2):
                off = (2 * pi) * CA
                fetch(1, off + CA)
                wait(0)
                compute(0, off)
                fetch(0, off + 2 * CA)
                wait(1)
                compute(1, off + CA)
                return c2
            lax.fori_loop(0, npair, pair_body, 0)
            wait(0)
            pltpu.sync_copy(al_v, alpha_h.at[pl.ds(sbase, SB)])
            return carry
        lax.fori_loop(0, nsup_w, sup_body, 0)

        pltpu.sync_copy(den_v, denp_h.at[wid])

    return kern(q, k, src, dst)


def _sc_agg(vcat, src, dst, alpha, den, zeros128, Epad, nsrc):
    CC = 64
    SB = 1024
    EperS = Epad // 16
    nsup = EperS // SB
    npair = SB // (2 * CC)
    rps = _NAP // 16
    mesh = plsc.VectorSubcoreMesh(core_axis_name="c", subcore_axis_name="s")

    @functools.partial(
        pl.kernel,
        out_type=jax.ShapeDtypeStruct((2 * _NAP, 128), _F32),
        mesh=mesh,
        compiler_params=pltpu.CompilerParams(
            use_tc_tiling_on_sc=False, needs_layout_passes=False),
        scratch_types=[
            pltpu.VMEM((SB + CC,), jnp.int32),
            pltpu.VMEM((SB + CC,), jnp.int32),
            pltpu.VMEM((SB,), _F32),
            pltpu.VMEM((2, CC), jnp.int32),
            pltpu.VMEM((CC, 128), _F32),
            pltpu.VMEM((CC, 128), _F32),
            pltpu.VMEM((CC, 128), _F32),
            pltpu.VMEM((CC, 128), _F32),
            pltpu.VMEM((_NAP,), _F32),
            pltpu.VMEM_SHARED((_NAP, 128), _F32),
            pltpu.SemaphoreType.DMA,
            pltpu.SemaphoreType.DMA,
            pltpu.SemaphoreType.DMA,
            pltpu.SemaphoreType.DMA,
        ],
    )
    def kern(vcat_h, src_h, dst_h, alpha_h, den_h, zero_h, out_h,
             src_v, dst_v, al_v, dst_sc, vr0, vr1, sc0, sc1, den_v, agg_sh,
             sem0, sem1, ssem0, ssem1):
        cid = lax.axis_index("c")
        sid = lax.axis_index("s")
        voff = cid * nsrc
        lane = lax.iota(jnp.int32, 16)

        for r in range(rps // 128):
            pltpu.sync_copy(zero_h, agg_sh.at[pl.ds(sid * rps + r * 128, 128)])
        pltpu.sync_copy(den_h, den_v)
        plsc.subcore_barrier()

        vbuf = (vr0, vr1)
        sems = (sem0, sem1)
        scbuf = (sc0, sc1)
        ssems = (ssem0, ssem1)

        def fetch(b, off):
            pltpu.async_copy(vcat_h.at[src_v.at[pl.ds(off, CC)]], vbuf[b],
                             sems[b])

        def wait(b):
            pltpu.make_async_copy(vcat_h.at[src_v.at[pl.ds(0, CC)]], vbuf[b],
                                  sems[b]).wait()

        def compute(b, off):
            pltpu.make_async_copy(scbuf[b], agg_sh.at[dst_sc.at[b]],
                                  ssems[b]).wait()
            vr = vbuf[b]
            sc_buf = scbuf[b]

            def grp_body(g, c2):
                a16 = al_v[pl.ds(off + g * 16, 16)]
                d16 = dst_v[pl.ds(off + g * 16, 16)]
                dst_sc[b, pl.ds(g * 16, 16)] = d16
                dg = plsc.load_gather(den_v, [d16])
                c16 = jnp.exp(a16) / (dg + 1e-16)
                for l in range(16):
                    e = g * 16 + l
                    cb = jnp.full((16,), c16[l], _F32)
                    for j in range(8):
                        sc_buf[e, pl.ds(j * 16, 16)] = vr[e, pl.ds(j * 16, 16)] * cb
                return c2
            lax.fori_loop(0, CC // 16, grp_body, 0)
            pltpu.async_copy(sc_buf, agg_sh.at[dst_sc.at[b]], ssems[b], add=True)

        def zb_body(i, carry):
            for j in range(8):
                sc0[i, pl.ds(j * 16, 16)] = jnp.zeros((16,), _F32)
                sc1[i, pl.ds(j * 16, 16)] = jnp.zeros((16,), _F32)
            return carry
        lax.fori_loop(0, CC, zb_body, 0)

        def zd_body(g, carry):
            dst_sc[0, pl.ds(g * 16, 16)] = jnp.zeros((16,), jnp.int32)
            dst_sc[1, pl.ds(g * 16, 16)] = jnp.zeros((16,), jnp.int32)
            return carry
        lax.fori_loop(0, CC // 16, zd_body, 0)
        pltpu.async_copy(sc0, agg_sh.at[dst_sc.at[0]], ssem0, add=True)
        pltpu.async_copy(sc1, agg_sh.at[dst_sc.at[1]], ssem1, add=True)

        s0 = sid * EperS

        def sup_body(si, carry):
            sbase = s0 + si * SB
            pltpu.sync_copy(src_h.at[pl.ds(sbase, SB + CC)], src_v)
            pltpu.sync_copy(dst_h.at[pl.ds(sbase, SB + CC)], dst_v)
            pltpu.sync_copy(alpha_h.at[pl.ds(sbase, SB)], al_v)

            def voff_body(g, c2):
                src_v[pl.ds(g * 16, 16)] = src_v[pl.ds(g * 16, 16)] + voff
                return c2
            lax.fori_loop(0, (SB + CC) // 16, voff_body, 0)

            fetch(0, 0)

            def pair_body(pi, c2):
                off = (2 * pi) * CC
                fetch(1, off + CC)
                wait(0)
                compute(0, off)
                fetch(0, off + 2 * CC)
                wait(1)
                compute(1, off + CC)
                return c2
            lax.fori_loop(0, npair, pair_body, 0)
            wait(0)
            return carry
        lax.fori_loop(0, nsup, sup_body, 0)

        pltpu.make_async_copy(sc0, agg_sh.at[dst_sc.at[0]], ssem0).wait()
        pltpu.make_async_copy(sc1, agg_sh.at[dst_sc.at[1]], ssem1).wait()

        plsc.subcore_barrier()
        pltpu.sync_copy(agg_sh.at[pl.ds(sid * rps, rps)],
                        out_h.at[pl.ds(cid * _NAP + sid * rps, rps)])

    return kern(vcat, src, dst, alpha, den, zeros128)


def _pad_edges(ei, Epad):
    E = ei.shape[1]
    src = jnp.concatenate([ei[0], jnp.zeros((Epad + 128 - E,), jnp.int32)])
    dst = jnp.concatenate([ei[1], jnp.full((Epad + 128 - E,), _NAP - 1, jnp.int32)])
    return src, dst


def _relation(h_src, h_dst_q, src, dst, Epad, Wq, bq, Wk, bk, Wv, bv, zeros128,
              bn_src, sup0=None):
    q, = _proj(h_dst_q, Wq, bq.reshape(1, _H), [_H], 1024)
    k, v0, v1 = _proj(h_src, jnp.concatenate([Wk, Wv], axis=1),
                      jnp.concatenate([bk, bv]).reshape(1, 2 * _H),
                      [_H, 128, 128], bn_src)
    vcat = jnp.concatenate([v0, v1], axis=0)
    alpha, denp = _sc_alpha_den(q, k, src, dst, Epad, sup0)
    den = _den_reduce(denp)
    return _sc_agg(vcat, src, dst, alpha, den, zeros128, Epad, h_src.shape[0])


def kernel(x_atom, x_bond, lin_aW, lin_ab, lin_bW, lin_bb, Wq, bq, Wk, bk,
           Wv, bv, Ws, bs, bn_g, bn_b, projW, projb, outW, outb,
           ei_ba, ei_aa, batch):
    NA = 10000
    xa = jnp.pad(x_atom, ((0, _NAP - NA), (0, 128 - x_atom.shape[1])))
    xb = jnp.pad(x_bond, ((0, 0), (0, 128 - x_bond.shape[1])))
    lin_aWp = jnp.pad(lin_aW, ((0, 128 - lin_aW.shape[0]), (0, 0)))
    lin_bWp = jnp.pad(lin_bW, ((0, 128 - lin_bW.shape[0]), (0, 0)))
    src_ba, dst_ba = _pad_edges(ei_ba, 32768)
    src_aa, dst_aa = _pad_edges(ei_aa, 163840)
    batch3 = batch.reshape(25, 1, 400)
    zeros128 = jnp.zeros((128, 128), _F32)
    outWp = jnp.pad(outW, ((0, 0), (0, 127)))
    outbp = jnp.pad(outb.reshape(1, 1), ((0, 0), (0, 127)))

    h_a, = _proj(xa, lin_aWp, lin_ab.reshape(1, _H), [_H], 1024)
    h_b, = _proj(xb, lin_bWp, lin_bb.reshape(1, _H), [_H], 1000)

    for l in range(3):
        aggcat1 = _relation(h_b, h_a, src_ba, dst_ba, 32768,
                            Wq[l, 0], bq[l, 0], Wk[l, 0], bk[l, 0],
                            Wv[l, 0], bv[l, 0], zeros128, 1000)
        aggcat2 = _relation(h_a, h_a, src_aa, dst_aa, 163840,
                            Wq[l, 1], bq[l, 1], Wk[l, 1], bk[l, 1],
                            Wv[l, 1], bv[l, 1], zeros128, 1024, sup0=8)
        o1, o2, sums = _stats(aggcat1, aggcat2, h_a,
                              Ws[l, 0], bs[l, 0].reshape(1, _H),
                              Ws[l, 1], bs[l, 1].reshape(1, _H))
        h_a = _norm_combine(o1, o2, sums,
                            bn_g[l, 0].reshape(1, _H), bn_b[l, 0].reshape(1, _H),
                            bn_g[l, 1].reshape(1, _H), bn_b[l, 1].reshape(1, _H))

    pooled, cnt2d = _pool(h_a, batch3)
    y = _head(pooled, cnt2d, projW, projb.reshape(1, _H), outWp, outbp)
    return y[:, :1]
